# baseline (speedup 1.0000x reference)
import functools

import jax
import jax.numpy as jnp
from jax import lax
from jax.experimental import pallas as pl
from jax.experimental.pallas import tpu as pltpu
from jax.experimental.pallas import tpu_sc as plsc

N = 10002
D = 128
HALF = 64
N_PAD = 10240
RPT = 640
C = 128
NTILES = 16
NCORES = 2
VPR = HALF // 16
DEPTH = 3


def _scale_chunk(gbuf, valsb, k):
    def group(g, carry):
        vv = valsb[k, pl.ds(g * 16, 16)]
        for l in range(16):
            e = g * 16 + l
            sval = vv[l]
            for j in range(VPR):
                sl = pl.ds(j * 16, 16)
                gbuf[e, sl] = gbuf[e, sl] * sval
        return carry
    lax.fori_loop(0, C // 16, group, 0)


def _body(final, nchunks, *refs):
    if final:
        (x_hbm, cols_hbm, rows_hbm, vals_hbm, x0_hbm, x1_hbm, out_hbm,
         acc, colsb, rowsb, valsb, *rest) = refs
    else:
        (x_hbm, cols_hbm, rows_hbm, vals_hbm, out_hbm,
         acc, colsb, rowsb, valsb, *rest) = refs
    gb = rest[0:DEPTH]
    sem_g = rest[DEPTH:2 * DEPTH]
    sem_s = rest[2 * DEPTH]

    c = lax.axis_index("c")
    s = lax.axis_index("s")

    def zrow(i, carry):
        for j in range(VPR):
            gb[0][i, pl.ds(j * 16, 16)] = jnp.zeros((16,), jnp.float32)
        return carry
    lax.fori_loop(0, C, zrow, 0)
    rbase = s * RPT
    for b in range(RPT // C):
        pltpu.sync_copy(gb[0], acc.at[pl.ds(rbase + b * C, C)])

    crow0 = (c * NTILES + s) * nchunks
    erow0 = s * nchunks
    pltpu.sync_copy(cols_hbm.at[pl.ds(crow0, nchunks)], colsb)
    pltpu.sync_copy(rows_hbm.at[pl.ds(erow0, nchunks)], rowsb)
    pltpu.sync_copy(vals_hbm.at[pl.ds(erow0, nchunks)], valsb)
    plsc.subcore_barrier()

    def issue_gather(k, slot):
        pltpu.async_copy(x_hbm.at[colsb.at[k]], gb[slot], sem_g[slot])

    def wait_gather(k, slot):
        pltpu.make_async_copy(x_hbm.at[colsb.at[k]], gb[slot],
                              sem_g[slot]).wait()

    def issue_scatter(k, slot):
        pltpu.async_copy(gb[slot], acc.at[rowsb.at[k]], sem_s, add=True)

    def wait_scatter(k, slot):
        pltpu.make_async_copy(gb[slot], acc.at[rowsb.at[k]],
                              sem_s).wait()

    niter = nchunks // DEPTH
    issue_gather(0, 0)
    issue_gather(1, 1)

    def iter_body(i, carry):
        for j in range(DEPTH):
            k = i * DEPTH + j
            nslot = (j + 2) % DEPTH
            wait_gather(k, j)
            _scale_chunk(gb[j], valsb, k)
            if j == 0:
                @pl.when(i > 0)
                def _():
                    wait_scatter(k - 1, (j + 2) % DEPTH)
            else:
                wait_scatter(k - 1, j - 1)
            issue_scatter(k, j)
            if j == 0:
                issue_gather(k + 2, nslot)
            else:
                @pl.when(i < niter - 1)
                def _():
                    issue_gather(k + 2, nslot)
        return carry
    lax.fori_loop(0, niter, iter_body, 0)
    wait_scatter(nchunks - 1, (nchunks - 1) % DEPTH)

    plsc.subcore_barrier()

    obase = c * N_PAD + rbase
    for b in range(RPT // C):
        r0 = rbase + b * C
        o0 = obase + b * C
        if not final:
            pltpu.sync_copy(acc.at[pl.ds(r0, C)], out_hbm.at[pl.ds(o0, C)])
        else:
            pltpu.sync_copy(acc.at[pl.ds(r0, C)], gb[0])
            pltpu.sync_copy(x0_hbm.at[pl.ds(o0, C)], gb[1])
            pltpu.sync_copy(x1_hbm.at[pl.ds(o0, C)], gb[2])

            def crow(i, carry):
                for j in range(VPR):
                    sl = pl.ds(j * 16, 16)
                    gb[0][i, sl] = (
                        gb[0][i, sl] + gb[1][i, sl] + gb[2][i, sl]
                    ) * (1.0 / 3.0)
                return carry
            lax.fori_loop(0, C, crow, 0)
            pltpu.sync_copy(gb[0], out_hbm.at[pl.ds(o0, C)])


def _make_kernel(nchunks, final):
    mesh = plsc.VectorSubcoreMesh(core_axis_name="c", subcore_axis_name="s")
    scratch = [
        pltpu.VMEM_SHARED((N_PAD, HALF), jnp.float32),
        pltpu.VMEM((nchunks, C), jnp.int32),
        pltpu.VMEM((nchunks, C), jnp.int32),
        pltpu.VMEM((nchunks, C), jnp.float32),
    ]
    scratch += [pltpu.VMEM((C, HALF), jnp.float32) for _ in range(DEPTH)]
    scratch += [pltpu.SemaphoreType.DMA for _ in range(DEPTH)]
    scratch += [pltpu.SemaphoreType.DMA]
    return pl.kernel(
        functools.partial(_body, final, nchunks),
        out_type=jax.ShapeDtypeStruct((2 * N_PAD, HALF), jnp.float32),
        mesh=mesh,
        scratch_types=scratch,
        compiler_params=pltpu.CompilerParams(use_tc_tiling_on_sc=False),
    )


def kernel(adjacency_indices, adjacency_values, embedding):
    rows = adjacency_indices[0].astype(jnp.int32)
    cols = adjacency_indices[1].astype(jnp.int32)
    vals = adjacency_values.astype(jnp.float32)
    e = vals.shape[0]
    ept = -(-(e // NTILES) // (DEPTH * C)) * (DEPTH * C)
    nchunks = ept // C
    e_pad = ept * NTILES

    cols_p = jnp.pad(cols, (0, e_pad - e))
    rows_p = jnp.pad(rows, (0, e_pad - e), constant_values=N)
    vals_p = jnp.pad(vals, (0, e_pad - e))
    cols2 = jnp.concatenate([cols_p, cols_p + N_PAD]).reshape(-1, C)
    rows2 = rows_p.reshape(-1, C)
    vals2 = vals_p.reshape(-1, C)

    emb_pad = jnp.pad(embedding.astype(jnp.float32),
                      ((0, N_PAD - N), (0, 0)))
    x0s = jnp.concatenate([emb_pad[:, :HALF], emb_pad[:, HALF:]], axis=0)

    layer_k = _make_kernel(nchunks, final=False)
    final_k = _make_kernel(nchunks, final=True)

    x1s = layer_k(x0s, cols2, rows2, vals2)
    outs = final_k(x1s, cols2, rows2, vals2, x0s, x1s)

    full = jnp.concatenate([outs[:N], outs[N_PAD:N_PAD + N]], axis=1)
    ds3 = N // 3
    return jnp.concatenate(
        [full[:ds3], full[ds3:2 * ds3], full[2 * ds3:]], axis=1)
